# baseline (speedup 1.0000x reference)
import jax
import jax.numpy as jnp
from jax import lax
from jax.experimental import pallas as pl
from jax.experimental.pallas import tpu as pltpu
from jax.experimental.pallas import tpu_sc as plsc

NC = 2
NS = 16
NW = NC * NS

CB = 256
NB = CB // 16


def _gather_body(idx_hbm, table_hbm, out_hbm, raw_v, pidx_v, pv_v, rowv_v,
                 pairs_v, rowsT_v, sem_g, sem_o, sem_i):
    L, B = idx_hbm.shape
    D = 64
    per_w = B // NW
    spc = per_w // CB
    n_chunks = L * spc
    wid = lax.axis_index("s") * NC + lax.axis_index("c")
    b0 = wid * per_w

    iota = lax.iota(jnp.int32, 16)
    for bb in range(NB):
        rowv_v[bb] = iota + bb * 16

    def chunk_l(c):
        return lax.div(c, spc)

    def chunk_boff(c):
        return b0 + lax.rem(c, spc) * CB

    def fetch_idx(c, p):
        cc = lax.rem(c, n_chunks)
        return pltpu.async_copy(
            idx_hbm.at[chunk_l(cc)].at[pl.ds(chunk_boff(cc), CB)],
            raw_v.at[p], sem_i,
        )

    def wait_idx(p):
        pltpu.make_async_copy(
            idx_hbm.at[0].at[pl.ds(0, CB)], raw_v.at[p], sem_i
        ).wait()

    def prep_idx(p):
        for k in range(NB):
            v = raw_v[p, pl.ds(k * 16, 16)]
            pidx_v[p, k // 8, pl.ds((k % 8) * 16, 16)] = (
                lax.shift_right_logical(v, 1))
            pv_v[p, k] = lax.shift_left(lax.bitwise_and(v, 1), 6)

    def issue_gather(p, _):
        for j in range(CB // 128):
            pltpu.async_copy(
                table_hbm.at[pidx_v.at[p].at[j]],
                pairs_v.at[p].at[pl.ds(j * 128, 128)],
                sem_g,
            )

    def drain_gather(p):
        pltpu.make_async_copy(
            table_hbm.at[pl.ds(0, CB)], pairs_v.at[p], sem_g
        ).wait()

    def transpose_chunk(p):
        pvec = iota * 0 + p

        def bb_body(bb, carry):
            rv = rowv_v[bb]
            col0 = pv_v[p, bb]
            for d in range(D):
                x16 = plsc.load_gather(pairs_v, [pvec, rv, col0 + d])
                rowsT_v[p, d, pl.ds(bb * 16, 16)] = x16
            return carry

        lax.fori_loop(0, NB, bb_body, 0)

    def issue_write(c, p):
        return pltpu.async_copy(
            rowsT_v.at[p],
            out_hbm.at[chunk_l(c)].at[slice(None), pl.ds(chunk_boff(c), CB)],
            sem_o,
        )

    def wait_write(p):
        pltpu.make_async_copy(
            rowsT_v.at[p], out_hbm.at[0].at[slice(None), pl.ds(0, CB)], sem_o
        ).wait()

    pltpu.sync_copy(idx_hbm.at[0].at[pl.ds(b0, CB)], raw_v.at[0])
    prep_idx(0)
    issue_gather(0, None)
    fetch_idx(1, 1)

    def body(i, carry):
        p = lax.rem(i, 2)
        q = 1 - p
        drain_gather(p)
        wait_idx(q)
        prep_idx(q)
        issue_gather(q, None)
        fetch_idx(i + 2, p)
        transpose_chunk(p)

        @pl.when(i > 0)
        def _():
            wait_write(q)

        issue_write(i, p)
        return carry

    lax.fori_loop(0, n_chunks - 1, body, 0)

    last = n_chunks - 1
    p = lax.rem(last, 2)
    drain_gather(p)
    wait_idx(1 - p)
    transpose_chunk(p)
    wait_write(1 - p)
    issue_write(last, p)
    wait_write(p)


def kernel(x, embedding):
    B, L = x.shape
    D = embedding.shape[1]
    V = embedding.shape[0]
    assert B % (NW * CB) == 0 and D == 64
    idx_t = x.T.astype(jnp.int32)
    table2 = embedding.reshape(V // 2, 2 * D)

    mesh = plsc.VectorSubcoreMesh(core_axis_name="c", subcore_axis_name="s")
    run = pl.kernel(
        _gather_body,
        out_type=jax.ShapeDtypeStruct((L, D, B), jnp.float32),
        mesh=mesh,
        scratch_types=[
            pltpu.VMEM((2, CB), jnp.int32),
            pltpu.VMEM((2, CB // 128, 128), jnp.int32),
            pltpu.VMEM((2, NB, 16), jnp.int32),
            pltpu.VMEM((NB, 16), jnp.int32),
            pltpu.VMEM((2, CB, 2 * D), jnp.float32),
            pltpu.VMEM((2, D, CB), jnp.float32),
            pltpu.SemaphoreType.DMA,
            pltpu.SemaphoreType.DMA,
            pltpu.SemaphoreType.DMA,
        ],
        compiler_params=pltpu.CompilerParams(
            use_tc_tiling_on_sc=True, needs_layout_passes=False
        ),
    )
    out_t = run(idx_t, table2)
    return jnp.transpose(out_t, (2, 0, 1))


# speedup vs baseline: 1.0053x; 1.0053x over previous
import jax
import jax.numpy as jnp
from jax import lax
from jax.experimental import pallas as pl
from jax.experimental.pallas import tpu as pltpu
from jax.experimental.pallas import tpu_sc as plsc

NC = 2
NS = 16
NW = NC * NS

CB = 256
NB = CB // 16


def _gather_body(idx_hbm, table_hbm, out_hbm,
                 raw0, raw1, pidx0, pidx1, pv0, pv1, rowv_v,
                 pairs0, pairs1, rowsT0, rowsT1, sem_g, sem_o, sem_i):
    L, B = idx_hbm.shape
    D = 64
    per_w = B // NW
    spc = per_w // CB
    n_chunks = L * spc
    wid = lax.axis_index("s") * NC + lax.axis_index("c")
    b0 = wid * per_w

    iota = lax.iota(jnp.int32, 16)
    for bb in range(NB):
        rowv_v[bb] = iota + bb * 16

    bufs = ((raw0, pidx0, pv0, pairs0, rowsT0),
            (raw1, pidx1, pv1, pairs1, rowsT1))

    def chunk_l(c):
        return lax.div(c, spc)

    def chunk_boff(c):
        return b0 + lax.rem(c, spc) * CB

    def fetch_idx(c, raw):
        cc = lax.rem(c, n_chunks)
        return pltpu.async_copy(
            idx_hbm.at[chunk_l(cc)].at[pl.ds(chunk_boff(cc), CB)],
            raw, sem_i,
        )

    def wait_idx(raw):
        pltpu.make_async_copy(
            idx_hbm.at[0].at[pl.ds(0, CB)], raw, sem_i
        ).wait()

    def prep_idx(raw, pidx, pv):
        for k in range(NB):
            v = raw[pl.ds(k * 16, 16)]
            pidx[k // 8, pl.ds((k % 8) * 16, 16)] = (
                lax.shift_right_logical(v, 1))
            pv[k] = lax.shift_left(lax.bitwise_and(v, 1), 6)

    def issue_gather(pidx, pairs):
        for j in range(CB // 128):
            pltpu.async_copy(
                table_hbm.at[pidx.at[j]],
                pairs.at[pl.ds(j * 128, 128)],
                sem_g,
            )

    def drain_gather(pairs):
        pltpu.make_async_copy(
            table_hbm.at[pl.ds(0, CB)], pairs, sem_g
        ).wait()

    def transpose_chunk(pairs, pv, rowsT):
        def bb_body(bb, carry):
            rv = rowv_v[bb]
            col0 = pv[bb]
            for d in range(D):
                x16 = plsc.load_gather(pairs, [rv, col0 + d])
                rowsT[d, pl.ds(bb * 16, 16)] = x16
            return carry

        lax.fori_loop(0, NB, bb_body, 0)

    def issue_write(c, rowsT):
        return pltpu.async_copy(
            rowsT,
            out_hbm.at[chunk_l(c)].at[slice(None), pl.ds(chunk_boff(c), CB)],
            sem_o,
        )

    def wait_write(rowsT):
        pltpu.make_async_copy(
            rowsT, out_hbm.at[0].at[slice(None), pl.ds(0, CB)], sem_o
        ).wait()

    pltpu.sync_copy(idx_hbm.at[0].at[pl.ds(b0, CB)], raw0)
    prep_idx(raw0, pidx0, pv0)
    issue_gather(pidx0, pairs0)
    fetch_idx(1, raw1)

    def step(i, P, Q):
        rawP, pidxP, pvP, pairsP, rowsTP = P
        rawQ, pidxQ, pvQ, pairsQ, rowsTQ = Q
        drain_gather(pairsP)
        wait_idx(rawQ)
        prep_idx(rawQ, pidxQ, pvQ)
        issue_gather(pidxQ, pairsQ)
        fetch_idx(i + 2, rawP)
        transpose_chunk(pairsP, pvP, rowsTP)

        @pl.when(i > 0)
        def _():
            wait_write(rowsTQ)

        issue_write(i, rowsTP)

    def body(i, carry):
        @pl.when(lax.rem(i, 2) == 0)
        def _():
            step(i, bufs[0], bufs[1])

        @pl.when(lax.rem(i, 2) == 1)
        def _():
            step(i, bufs[1], bufs[0])

        return carry

    lax.fori_loop(0, n_chunks - 1, body, 0)

    last = n_chunks - 1
    P, Q = bufs[(n_chunks - 1) % 2], bufs[n_chunks % 2]
    drain_gather(P[3])
    wait_idx(Q[0])
    transpose_chunk(P[3], P[2], P[4])
    wait_write(Q[4])
    issue_write(last, P[4])
    wait_write(P[4])


def kernel(x, embedding):
    B, L = x.shape
    D = embedding.shape[1]
    V = embedding.shape[0]
    assert B % (NW * CB) == 0 and D == 64
    idx_t = x.T.astype(jnp.int32)
    table2 = embedding.reshape(V // 2, 2 * D)

    mesh = plsc.VectorSubcoreMesh(core_axis_name="c", subcore_axis_name="s")
    run = pl.kernel(
        _gather_body,
        out_type=jax.ShapeDtypeStruct((L, D, B), jnp.float32),
        mesh=mesh,
        scratch_types=[
            pltpu.VMEM((CB,), jnp.int32),
            pltpu.VMEM((CB,), jnp.int32),
            pltpu.VMEM((CB // 128, 128), jnp.int32),
            pltpu.VMEM((CB // 128, 128), jnp.int32),
            pltpu.VMEM((NB, 16), jnp.int32),
            pltpu.VMEM((NB, 16), jnp.int32),
            pltpu.VMEM((NB, 16), jnp.int32),
            pltpu.VMEM((CB, 2 * D), jnp.float32),
            pltpu.VMEM((CB, 2 * D), jnp.float32),
            pltpu.VMEM((D, CB), jnp.float32),
            pltpu.VMEM((D, CB), jnp.float32),
            pltpu.SemaphoreType.DMA,
            pltpu.SemaphoreType.DMA,
            pltpu.SemaphoreType.DMA,
        ],
        compiler_params=pltpu.CompilerParams(
            use_tc_tiling_on_sc=True, needs_layout_passes=False
        ),
    )
    out_t = run(idx_t, table2)
    return jnp.transpose(out_t, (2, 0, 1))


# speedup vs baseline: 1.1531x; 1.1470x over previous
import jax
import jax.numpy as jnp
from jax import lax
from jax.experimental import pallas as pl
from jax.experimental.pallas import tpu as pltpu
from jax.experimental.pallas import tpu_sc as plsc

NC = 2
NS = 16
NW = NC * NS

CB = 256
NB = CB // 16


def _gather_body(idx_hbm, table_hbm, out_hbm,
                 raw0, raw1, pidx0, pidx1, pv0, pv1, rowv_v,
                 pairs0, pairs1, rowsT0, rowsT1, sem_g, sem_o, sem_i):
    L, B = idx_hbm.shape
    D = 64
    per_w = B // NW
    spc = per_w // CB
    n_chunks = L * spc
    wid = lax.axis_index("s") * NC + lax.axis_index("c")
    b0 = wid * per_w

    iota = lax.iota(jnp.int32, 16)
    for bb in range(NB):
        rowv_v[bb] = iota + bb * 16

    bufs = ((raw0, pidx0, pv0, pairs0, rowsT0),
            (raw1, pidx1, pv1, pairs1, rowsT1))

    def chunk_l(c):
        return lax.div(c, spc)

    def chunk_boff(c):
        return b0 + lax.rem(c, spc) * CB

    def fetch_idx(c, raw):
        cc = lax.rem(c, n_chunks)
        return pltpu.async_copy(
            idx_hbm.at[chunk_l(cc)].at[pl.ds(chunk_boff(cc), CB)],
            raw, sem_i,
        )

    def wait_idx(raw):
        pltpu.make_async_copy(
            idx_hbm.at[0].at[pl.ds(0, CB)], raw, sem_i
        ).wait()

    def prep_idx(raw, pidx, pv):
        for k in range(NB):
            v = raw[pl.ds(k * 16, 16)]
            pidx[k // 8, pl.ds((k % 8) * 16, 16)] = (
                lax.shift_right_logical(v, 1))
            pv[k] = lax.shift_left(lax.bitwise_and(v, 1), 6)

    def issue_gather(pidx, pairs):
        for j in range(CB // 128):
            pltpu.async_copy(
                table_hbm.at[pidx.at[j]],
                pairs.at[pl.ds(j * 128, 128)],
                sem_g,
            )

    def drain_gather(pairs):
        pltpu.make_async_copy(
            table_hbm.at[pl.ds(0, CB)], pairs, sem_g
        ).wait()

    def transpose_chunk(pairs, pv, rowsT):
        @plsc.parallel_loop(0, NB, 1, unroll=2)
        def bb_body(bb):
            rv = rowv_v[bb]
            col0 = pv[bb]
            for d in range(D):
                x16 = plsc.load_gather(pairs, [rv, col0 + d])
                rowsT[d, pl.ds(bb * 16, 16)] = x16

    def issue_write(c, rowsT):
        return pltpu.async_copy(
            rowsT,
            out_hbm.at[chunk_l(c)].at[slice(None), pl.ds(chunk_boff(c), CB)],
            sem_o,
        )

    def wait_write(rowsT):
        pltpu.make_async_copy(
            rowsT, out_hbm.at[0].at[slice(None), pl.ds(0, CB)], sem_o
        ).wait()

    pltpu.sync_copy(idx_hbm.at[0].at[pl.ds(b0, CB)], raw0)
    prep_idx(raw0, pidx0, pv0)
    issue_gather(pidx0, pairs0)
    fetch_idx(1, raw1)

    def step(i, P, Q):
        rawP, pidxP, pvP, pairsP, rowsTP = P
        rawQ, pidxQ, pvQ, pairsQ, rowsTQ = Q
        drain_gather(pairsP)
        wait_idx(rawQ)
        prep_idx(rawQ, pidxQ, pvQ)
        issue_gather(pidxQ, pairsQ)
        fetch_idx(i + 2, rawP)
        transpose_chunk(pairsP, pvP, rowsTP)

        @pl.when(i > 0)
        def _():
            wait_write(rowsTQ)

        issue_write(i, rowsTP)

    def body(i, carry):
        @pl.when(lax.rem(i, 2) == 0)
        def _():
            step(i, bufs[0], bufs[1])

        @pl.when(lax.rem(i, 2) == 1)
        def _():
            step(i, bufs[1], bufs[0])

        return carry

    lax.fori_loop(0, n_chunks - 1, body, 0)

    last = n_chunks - 1
    P, Q = bufs[(n_chunks - 1) % 2], bufs[n_chunks % 2]
    drain_gather(P[3])
    wait_idx(Q[0])
    transpose_chunk(P[3], P[2], P[4])
    wait_write(Q[4])
    issue_write(last, P[4])
    wait_write(P[4])


def kernel(x, embedding):
    B, L = x.shape
    D = embedding.shape[1]
    V = embedding.shape[0]
    assert B % (NW * CB) == 0 and D == 64
    idx_t = x.T.astype(jnp.int32)
    table2 = embedding.reshape(V // 2, 2 * D)

    mesh = plsc.VectorSubcoreMesh(core_axis_name="c", subcore_axis_name="s")
    run = pl.kernel(
        _gather_body,
        out_type=jax.ShapeDtypeStruct((L, D, B), jnp.float32),
        mesh=mesh,
        scratch_types=[
            pltpu.VMEM((CB,), jnp.int32),
            pltpu.VMEM((CB,), jnp.int32),
            pltpu.VMEM((CB // 128, 128), jnp.int32),
            pltpu.VMEM((CB // 128, 128), jnp.int32),
            pltpu.VMEM((NB, 16), jnp.int32),
            pltpu.VMEM((NB, 16), jnp.int32),
            pltpu.VMEM((NB, 16), jnp.int32),
            pltpu.VMEM((CB, 2 * D), jnp.float32),
            pltpu.VMEM((CB, 2 * D), jnp.float32),
            pltpu.VMEM((D, CB), jnp.float32),
            pltpu.VMEM((D, CB), jnp.float32),
            pltpu.SemaphoreType.DMA,
            pltpu.SemaphoreType.DMA,
            pltpu.SemaphoreType.DMA,
        ],
        compiler_params=pltpu.CompilerParams(
            use_tc_tiling_on_sc=True, needs_layout_passes=False
        ),
    )
    out_t = run(idx_t, table2)
    return jnp.transpose(out_t, (2, 0, 1))


# speedup vs baseline: 1.1574x; 1.0037x over previous
import jax
import jax.numpy as jnp
from jax import lax
from jax.experimental import pallas as pl
from jax.experimental.pallas import tpu as pltpu
from jax.experimental.pallas import tpu_sc as plsc

NC = 2
NS = 16
NW = NC * NS

CB = 256
NB = CB // 16


def _gather_body(idx_hbm, table_hbm, out_hbm,
                 raw0, raw1, pidx0, pidx1, pv0, pv1, rowv_v,
                 pairs0, pairs1, rowsT0, rowsT1, sem_g, sem_o, sem_i):
    L, B = idx_hbm.shape
    D = 64
    per_w = B // NW
    spc = per_w // CB
    n_chunks = L * spc
    wid = lax.axis_index("s") * NC + lax.axis_index("c")
    b0 = wid * per_w

    iota = lax.iota(jnp.int32, 16)
    for bb in range(NB):
        rowv_v[bb] = iota + bb * 16

    bufs = ((raw0, pidx0, pv0, pairs0, rowsT0),
            (raw1, pidx1, pv1, pairs1, rowsT1))

    def chunk_l(c):
        return lax.div(c, spc)

    def chunk_boff(c):
        return b0 + lax.rem(c, spc) * CB

    def fetch_idx(c, raw):
        cc = lax.rem(c, n_chunks)
        return pltpu.async_copy(
            idx_hbm.at[chunk_l(cc)].at[pl.ds(chunk_boff(cc), CB)],
            raw, sem_i,
        )

    def wait_idx(raw):
        pltpu.make_async_copy(
            idx_hbm.at[0].at[pl.ds(0, CB)], raw, sem_i
        ).wait()

    def prep_idx(raw, pidx, pv):
        for k in range(NB):
            v = raw[pl.ds(k * 16, 16)]
            pidx[k // 8, pl.ds((k % 8) * 16, 16)] = (
                lax.shift_right_logical(v, 1))
            pv[k] = lax.shift_left(lax.bitwise_and(v, 1), 6)

    def issue_gather(pidx, pairs):
        for j in range(CB // 128):
            pltpu.async_copy(
                table_hbm.at[pidx.at[j]],
                pairs.at[pl.ds(j * 128, 128)],
                sem_g,
            )

    def drain_gather(pairs):
        pltpu.make_async_copy(
            table_hbm.at[pl.ds(0, CB)], pairs, sem_g
        ).wait()

    def transpose_chunk(pairs, pv, rowsT):
        @plsc.parallel_loop(0, NB, 1, unroll=2)
        def bb_body(bb):
            rv = rowv_v[bb]
            col0 = pv[bb]
            for d in range(D):
                x16 = plsc.load_gather(pairs, [rv, col0 + d])
                rowsT[d, pl.ds(bb * 16, 16)] = x16

    def issue_write(c, rowsT):
        return pltpu.async_copy(
            rowsT,
            out_hbm.at[chunk_l(c)].at[slice(None), pl.ds(chunk_boff(c), CB)],
            sem_o,
        )

    def wait_write(rowsT):
        pltpu.make_async_copy(
            rowsT, out_hbm.at[0].at[slice(None), pl.ds(0, CB)], sem_o
        ).wait()

    pltpu.sync_copy(idx_hbm.at[0].at[pl.ds(b0, CB)], raw0)
    prep_idx(raw0, pidx0, pv0)
    issue_gather(pidx0, pairs0)
    fetch_idx(1, raw1)

    def step(i, P, Q):
        rawP, pidxP, pvP, pairsP, rowsTP = P
        rawQ, pidxQ, pvQ, pairsQ, rowsTQ = Q
        drain_gather(pairsP)
        wait_idx(rawQ)
        prep_idx(rawQ, pidxQ, pvQ)
        issue_gather(pidxQ, pairsQ)
        fetch_idx(i + 2, rawP)
        transpose_chunk(pairsP, pvP, rowsTP)

        @pl.when(i > 0)
        def _():
            wait_write(rowsTQ)

        issue_write(i, rowsTP)

    def body(i, carry):
        @pl.when(lax.rem(i, 2) == 0)
        def _():
            step(i, bufs[0], bufs[1])

        @pl.when(lax.rem(i, 2) == 1)
        def _():
            step(i, bufs[1], bufs[0])

        return carry

    lax.fori_loop(0, n_chunks - 1, body, 0)

    last = n_chunks - 1
    P, Q = bufs[(n_chunks - 1) % 2], bufs[n_chunks % 2]
    drain_gather(P[3])
    wait_idx(Q[0])
    transpose_chunk(P[3], P[2], P[4])
    wait_write(Q[4])
    issue_write(last, P[4])
    wait_write(P[4])


def kernel(x, embedding):
    B, L = x.shape
    D = embedding.shape[1]
    V = embedding.shape[0]
    assert B % (NW * CB) == 0 and D == 64
    idx_t = x.T.astype(jnp.int32)
    table2 = embedding.reshape(V // 2, 2 * D)

    mesh = plsc.VectorSubcoreMesh(core_axis_name="c", subcore_axis_name="s")
    run = pl.kernel(
        _gather_body,
        out_type=jax.ShapeDtypeStruct((L, D, B), jnp.float32),
        mesh=mesh,
        scratch_types=[
            pltpu.VMEM((CB,), jnp.int32),
            pltpu.VMEM((CB,), jnp.int32),
            pltpu.VMEM((CB // 128, 128), jnp.int32),
            pltpu.VMEM((CB // 128, 128), jnp.int32),
            pltpu.VMEM((NB, 16), jnp.int32),
            pltpu.VMEM((NB, 16), jnp.int32),
            pltpu.VMEM((NB, 16), jnp.int32),
            pltpu.VMEM((CB, 2 * D), jnp.float32),
            pltpu.VMEM((CB, 2 * D), jnp.float32),
            pltpu.VMEM((D, CB), jnp.float32),
            pltpu.VMEM((D, CB), jnp.float32),
            pltpu.SemaphoreType.DMA,
            pltpu.SemaphoreType.DMA,
            pltpu.SemaphoreType.DMA,
        ],
        compiler_params=pltpu.CompilerParams(
            use_tc_tiling_on_sc=True, needs_layout_passes=False,
            disable_bounds_checks=True,
        ),
    )
    out_t = run(idx_t, table2)
    return jnp.transpose(out_t, (2, 0, 1))


# speedup vs baseline: 2.1838x; 1.8868x over previous
import jax
import jax.numpy as jnp
from jax import lax
from jax.experimental import pallas as pl
from jax.experimental.pallas import tpu as pltpu
from jax.experimental.pallas import tpu_sc as plsc

NC = 2
NS = 16
NW = NC * NS


def _gather_body(idx_hbm, table_hbm, out_hbm, idx_v, rows_v, sem_g, sem_o,
                 sem_i):
    L, B = idx_hbm.shape
    D = table_hbm.shape[1]
    CB = B // NW
    wid = lax.axis_index("s") * NC + lax.axis_index("c")
    b0 = wid * CB

    def issue_gather(p, _):
        pltpu.async_copy(table_hbm.at[idx_v.at[p]], rows_v.at[p], sem_g)

    def drain_gather(p):
        pltpu.make_async_copy(
            table_hbm.at[pl.ds(0, CB)], rows_v.at[p], sem_g
        ).wait()

    def fetch_idx(l, p):
        return pltpu.async_copy(
            idx_hbm.at[lax.rem(l, L)].at[pl.ds(b0, CB)], idx_v.at[p], sem_i
        )

    def wait_idx(p):
        pltpu.make_async_copy(
            idx_hbm.at[0].at[pl.ds(0, CB)], idx_v.at[p], sem_i
        ).wait()

    def issue_write(l, p):
        return pltpu.async_copy(
            rows_v.at[p],
            out_hbm.at[pl.ds(b0, CB), pl.ds(l * D, D)],
            sem_o,
        )

    def wait_write(p):
        pltpu.make_async_copy(
            rows_v.at[p], out_hbm.at[pl.ds(0, CB), pl.ds(0, D)], sem_o
        ).wait()

    pltpu.sync_copy(idx_hbm.at[0].at[pl.ds(b0, CB)], idx_v.at[0])
    issue_gather(0, None)
    fetch_idx(1, 1)

    def body(l, carry):
        p = lax.rem(l, 2)
        q = 1 - p
        drain_gather(p)

        @pl.when(l > 0)
        def _():
            wait_write(q)

        issue_write(l, p)
        wait_idx(q)
        fetch_idx(l + 2, p)
        issue_gather(q, None)
        return carry

    lax.fori_loop(0, L - 1, body, 0)

    last = L - 1
    p = lax.rem(last, 2)
    drain_gather(p)
    wait_write(1 - p)
    issue_write(last, p)
    wait_idx(1 - p)
    wait_write(p)


def kernel(x, embedding):
    B, L = x.shape
    D = embedding.shape[1]
    assert B % NW == 0
    CB = B // NW
    idx_t = x.T.astype(jnp.int32)

    mesh = plsc.VectorSubcoreMesh(core_axis_name="c", subcore_axis_name="s")
    run = pl.kernel(
        _gather_body,
        out_type=jax.ShapeDtypeStruct((B, L * D), jnp.float32),
        mesh=mesh,
        scratch_types=[
            pltpu.VMEM((2, CB), jnp.int32),
            pltpu.VMEM((2, CB, D), jnp.float32),
            pltpu.SemaphoreType.DMA,
            pltpu.SemaphoreType.DMA,
            pltpu.SemaphoreType.DMA,
        ],
        compiler_params=pltpu.CompilerParams(use_tc_tiling_on_sc=False),
    )
    out = run(idx_t, embedding)
    return out.reshape(B, L, D)
